# Initial kernel scaffold; baseline (speedup 1.0000x reference)
import jax
import jax.numpy as jnp
from jax.experimental import pallas as pl


def kernel(x, edge_indexw, edge_indexm, edge_attrw, edge_attrm, W0, b0, W1, b1, W2, b2):
    raise NotImplementedError("write your pallas kernel here")


# speedup vs baseline: 2.5228x; 2.5228x over previous
import jax
import jax.numpy as jnp
from jax import lax
from jax.experimental import pallas as pl
from jax.experimental.pallas import tpu as pltpu
from jax.experimental.pallas import tpu_sc as plsc

N_NODES = 10000
N_EDGES = 320000
D = 128

NC = 2
NS = 16

CHUNK = 100
NCHUNKS = N_EDGES // CHUNK
ROWS_PER_TILE = NCHUNKS // NS

NODE_BLK = 632
NODE_BLK_LAST = N_NODES - NODE_BLK * (NS - 1)
ZBLK = 96


def _zero_acc_range(acc, rows_v, base, count):
    nfull = count // ZBLK
    tail = count - nfull * ZBLK
    for k in range(nfull):
        pltpu.sync_copy(rows_v.at[pl.ds(0, ZBLK)],
                        acc.at[pl.ds(base + k * ZBLK, ZBLK)])
    if tail:
        pltpu.sync_copy(rows_v.at[pl.ds(0, tail)],
                        acc.at[pl.ds(base + nfull * ZBLK, tail)])


def _segsum_body(destw_hbm, destm_hbm, attrw_hbm, attrm_hbm,
                 outw_hbm, outm_hbm, acc, idx_v, rows_v):
    c = lax.axis_index("c")
    s = lax.axis_index("s")

    def zero_row(r, _):
        for k in range(D // 16):
            rows_v[r, pl.ds(k * 16, 16)] = jnp.zeros((16,), jnp.float32)
        return 0
    lax.fori_loop(0, CHUNK, zero_row, 0)

    @pl.when(s < NS - 1)
    def _():
        _zero_acc_range(acc, rows_v, s * NODE_BLK, NODE_BLK)

    @pl.when(s == NS - 1)
    def _():
        _zero_acc_range(acc, rows_v, (NS - 1) * NODE_BLK, NODE_BLK_LAST)

    plsc.subcore_barrier()

    rbase = s * ROWS_PER_TILE

    def scatter_edges(dest_hbm, attr_hbm):
        pltpu.sync_copy(dest_hbm.at[pl.ds(rbase, ROWS_PER_TILE)], idx_v)

        def chunk_body(j, _):
            pltpu.sync_copy(attr_hbm.at[rbase + j], rows_v)
            pltpu.sync_copy(rows_v, acc.at[idx_v.at[j]], add=True)
            return 0
        lax.fori_loop(0, ROWS_PER_TILE, chunk_body, 0)

    @pl.when(c == 0)
    def _():
        scatter_edges(destw_hbm, attrw_hbm)

    @pl.when(c == 1)
    def _():
        scatter_edges(destm_hbm, attrm_hbm)

    plsc.subcore_barrier()

    def writeout(out_hbm):
        @pl.when(s < NS - 1)
        def _():
            pltpu.sync_copy(acc.at[pl.ds(s * NODE_BLK, NODE_BLK)],
                            out_hbm.at[pl.ds(s * NODE_BLK, NODE_BLK)])

        @pl.when(s == NS - 1)
        def _():
            pltpu.sync_copy(acc.at[pl.ds((NS - 1) * NODE_BLK, NODE_BLK_LAST)],
                            out_hbm.at[pl.ds((NS - 1) * NODE_BLK, NODE_BLK_LAST)])

    @pl.when(c == 0)
    def _():
        writeout(outw_hbm)

    @pl.when(c == 1)
    def _():
        writeout(outm_hbm)


def _segment_sums(destw, destm, attrw, attrm):
    mesh = plsc.VectorSubcoreMesh(core_axis_name="c", subcore_axis_name="s",
                                  num_cores=NC, num_subcores=NS)
    f = pl.kernel(
        _segsum_body,
        out_type=(jax.ShapeDtypeStruct((N_NODES, D), jnp.float32),
                  jax.ShapeDtypeStruct((N_NODES, D), jnp.float32)),
        mesh=mesh,
        scratch_types=[
            pltpu.VMEM_SHARED((N_NODES, D), jnp.float32),
            pltpu.VMEM((ROWS_PER_TILE, CHUNK), jnp.int32),
            pltpu.VMEM((CHUNK, D), jnp.float32),
        ],
    )
    return f(destw, destm, attrw, attrm)


def _mlp_body(x_ref, ow_ref, om_ref, w0x_ref, w0w_ref, w0m_ref, b0_ref,
              w1_ref, b1_ref, w2_ref, b2_ref, out_ref):
    def silu(h):
        return h * (1.0 / (1.0 + jnp.exp(-h)))
    h = (jnp.dot(x_ref[...], w0x_ref[...], preferred_element_type=jnp.float32)
         + jnp.dot(ow_ref[...], w0w_ref[...], preferred_element_type=jnp.float32)
         + jnp.dot(om_ref[...], w0m_ref[...], preferred_element_type=jnp.float32)
         + b0_ref[...])
    h = silu(h)
    h = silu(jnp.dot(h, w1_ref[...], preferred_element_type=jnp.float32) + b1_ref[...])
    out_ref[...] = (jnp.dot(h, w2_ref[...], preferred_element_type=jnp.float32)
                    + b2_ref[...])


def _mlp(x, outw, outm, W0, b0, W1, b1, W2, b2):
    blk = 1000
    grid = (N_NODES // blk,)
    row_spec = pl.BlockSpec((blk, D), lambda i: (i, 0))
    w_spec = pl.BlockSpec((D, D), lambda i: (0, 0))
    b_spec = pl.BlockSpec((1, D), lambda i: (0, 0))
    return pl.pallas_call(
        _mlp_body,
        grid=grid,
        in_specs=[row_spec, row_spec, row_spec,
                  w_spec, w_spec, w_spec, b_spec,
                  w_spec, b_spec, w_spec, b_spec],
        out_specs=row_spec,
        out_shape=jax.ShapeDtypeStruct((N_NODES, D), jnp.float32),
    )(x, outw, outm, W0[0:D], W0[D:2 * D], W0[2 * D:3 * D], b0.reshape(1, D),
      W1, b1.reshape(1, D), W2, b2.reshape(1, D))


def kernel(x, edge_indexw, edge_indexm, edge_attrw, edge_attrm,
           W0, b0, W1, b1, W2, b2):
    destw = edge_indexw[1].astype(jnp.int32).reshape(NCHUNKS, CHUNK)
    destm = edge_indexm[1].astype(jnp.int32).reshape(NCHUNKS, CHUNK)
    attrw = edge_attrw.reshape(NCHUNKS, CHUNK, D)
    attrm = edge_attrm.reshape(NCHUNKS, CHUNK, D)
    outw, outm = _segment_sums(destw, destm, attrw, attrm)
    return _mlp(x, outw, outm, W0, b0, W1, b1, W2, b2)
